# Initial kernel scaffold; baseline (speedup 1.0000x reference)
import jax
import jax.numpy as jnp
from jax.experimental import pallas as pl


def kernel(table, idx):
    raise NotImplementedError("write your pallas kernel here")


# speedup vs baseline: 303.3292x; 303.3292x over previous
import functools

import jax
import jax.numpy as jnp
from jax import lax
from jax.experimental import pallas as pl
from jax.experimental.pallas import tpu as pltpu
from jax.experimental.pallas import tpu_sc as plsc

_WIN = 32
_B = (_WIN * _WIN) ** 2
_TBL = (2 * _WIN - 1) ** 2
_TBL_PAD = 4032
_NC, _NS, _L = 2, 16, 16
_NW = _NC * _NS
_BPW = _B // _NW
_GROUPS = _BPW // _L
_UNROLL = 8


@functools.partial(
    pl.kernel,
    out_type=jax.ShapeDtypeStruct((_B,), jnp.float32),
    mesh=plsc.VectorSubcoreMesh(
        core_axis_name="c", subcore_axis_name="s",
        num_cores=_NC, num_subcores=_NS,
    ),
    compiler_params=pltpu.CompilerParams(needs_layout_passes=False),
    scratch_types=[
        pltpu.VMEM((_TBL_PAD,), jnp.float32),
        pltpu.VMEM((_BPW,), jnp.int32),
        pltpu.VMEM((_BPW,), jnp.float32),
    ],
)
def _sc_gather(table_hbm, idx_hbm, out_hbm, table_v, idx_v, out_v):
    wid = lax.axis_index("s") * _NC + lax.axis_index("c")
    base = wid * _BPW
    pltpu.sync_copy(table_hbm, table_v)
    pltpu.sync_copy(idx_hbm.at[pl.ds(base, _BPW)], idx_v)

    def body(i, carry):
        for u in range(_UNROLL):
            off = (i * _UNROLL + u) * _L
            iv = idx_v[pl.ds(off, _L)]
            out_v[pl.ds(off, _L)] = plsc.load_gather(table_v, [iv])
        return carry

    lax.fori_loop(0, _GROUPS // _UNROLL, body, 0)
    pltpu.sync_copy(out_v, out_hbm.at[pl.ds(base, _BPW)])


def kernel(table, idx):
    tbl = jnp.zeros((_TBL_PAD,), jnp.float32).at[:_TBL].set(table.reshape(-1))
    out = _sc_gather(tbl, idx.reshape(-1))
    return out.reshape(idx.shape)


# speedup vs baseline: 303.5215x; 1.0006x over previous
import functools

import jax
import jax.numpy as jnp
from jax import lax
from jax.experimental import pallas as pl
from jax.experimental.pallas import tpu as pltpu
from jax.experimental.pallas import tpu_sc as plsc

_WIN = 32
_B = (_WIN * _WIN) ** 2
_TBL = (2 * _WIN - 1) ** 2
_NC, _NS, _L = 2, 16, 16
_NW = _NC * _NS
_BPW = _B // _NW
_CH = 8192
_NCH = _BPW // _CH
_UNROLL = 8


@functools.partial(
    pl.kernel,
    out_type=jax.ShapeDtypeStruct((_B,), jnp.float32),
    mesh=plsc.VectorSubcoreMesh(
        core_axis_name="c", subcore_axis_name="s",
        num_cores=_NC, num_subcores=_NS,
    ),
    compiler_params=pltpu.CompilerParams(needs_layout_passes=False),
    scratch_types=[
        pltpu.VMEM((_TBL,), jnp.float32),
        pltpu.VMEM((2, _CH), jnp.int32),
        pltpu.VMEM((2, _CH), jnp.float32),
        pltpu.SemaphoreType.DMA,
        pltpu.SemaphoreType.DMA,
        pltpu.SemaphoreType.DMA,
        pltpu.SemaphoreType.DMA,
    ],
)
def _sc_gather(table_hbm, idx_hbm, out_hbm, table_v, idx_v, out_v,
               isem0, isem1, osem0, osem1):
    isems = (isem0, isem1)
    osems = (osem0, osem1)
    wid = lax.axis_index("s") * _NC + lax.axis_index("c")
    base = wid * _BPW

    pltpu.async_copy(idx_hbm.at[pl.ds(base, _CH)], idx_v.at[0], isems[0])
    pltpu.sync_copy(table_hbm, table_v)

    for k in range(_NCH):
        b = k % 2
        if k + 1 < _NCH:
            pltpu.async_copy(idx_hbm.at[pl.ds(base + (k + 1) * _CH, _CH)],
                             idx_v.at[(k + 1) % 2], isems[(k + 1) % 2])
        pltpu.make_async_copy(idx_hbm.at[pl.ds(base + k * _CH, _CH)],
                              idx_v.at[b], isems[b]).wait()
        if k >= 2:
            pltpu.make_async_copy(out_v.at[b],
                                  out_hbm.at[pl.ds(base + (k - 2) * _CH, _CH)],
                                  osems[b]).wait()

        def body(i, carry, b=b):
            for u in range(_UNROLL):
                off = (i * _UNROLL + u) * _L
                iv = idx_v[b, pl.ds(off, _L)]
                out_v[b, pl.ds(off, _L)] = plsc.load_gather(table_v, [iv])
            return carry

        lax.fori_loop(0, _CH // _L // _UNROLL, body, 0)
        pltpu.async_copy(out_v.at[b], out_hbm.at[pl.ds(base + k * _CH, _CH)],
                         osems[b])

    pltpu.make_async_copy(out_v.at[(_NCH - 2) % 2],
                          out_hbm.at[pl.ds(base + (_NCH - 2) * _CH, _CH)],
                          osems[(_NCH - 2) % 2]).wait()
    pltpu.make_async_copy(out_v.at[(_NCH - 1) % 2],
                          out_hbm.at[pl.ds(base + (_NCH - 1) * _CH, _CH)],
                          osems[(_NCH - 1) % 2]).wait()


def kernel(table, idx):
    out = _sc_gather(table.reshape(-1), idx.reshape(-1))
    return out.reshape(idx.shape)


# speedup vs baseline: 353.2164x; 1.1637x over previous
import functools

import jax
import jax.numpy as jnp
from jax import lax
from jax.experimental import pallas as pl
from jax.experimental.pallas import tpu as pltpu
from jax.experimental.pallas import tpu_sc as plsc

_WIN = 32
_B = (_WIN * _WIN) ** 2
_TBL = (2 * _WIN - 1) ** 2
_NC, _NS, _L = 2, 16, 16
_NW = _NC * _NS
_BPW = _B // _NW
_CH = 8192
_NCH = _BPW // _CH
_UNROLL = 8


@functools.partial(
    pl.kernel,
    out_type=jax.ShapeDtypeStruct((_B,), jnp.float32),
    mesh=plsc.VectorSubcoreMesh(
        core_axis_name="c", subcore_axis_name="s",
        num_cores=_NC, num_subcores=_NS,
    ),
    compiler_params=pltpu.CompilerParams(needs_layout_passes=False),
    scratch_types=[
        pltpu.VMEM((_TBL,), jnp.float32),
        pltpu.VMEM((2, _CH), jnp.int32),
        pltpu.VMEM((2, _CH), jnp.float32),
        pltpu.SemaphoreType.DMA,
        pltpu.SemaphoreType.DMA,
        pltpu.SemaphoreType.DMA,
        pltpu.SemaphoreType.DMA,
    ],
)
def _sc_gather(table_hbm, idx_hbm, out_hbm, table_v, idx_v, out_v,
               isem0, isem1, osem0, osem1):
    isems = (isem0, isem1)
    osems = (osem0, osem1)
    wid = lax.axis_index("s") * _NC + lax.axis_index("c")
    base = wid * _BPW

    pltpu.async_copy(idx_hbm.at[pl.ds(base, _CH)], idx_v.at[0], isems[0])
    pltpu.sync_copy(table_hbm, table_v)

    for k in range(_NCH):
        b = k % 2
        if k + 1 < _NCH:
            pltpu.async_copy(idx_hbm.at[pl.ds(base + (k + 1) * _CH, _CH)],
                             idx_v.at[(k + 1) % 2], isems[(k + 1) % 2])
        pltpu.make_async_copy(idx_hbm.at[pl.ds(base + k * _CH, _CH)],
                              idx_v.at[b], isems[b]).wait()
        if k >= 2:
            pltpu.make_async_copy(out_v.at[b],
                                  out_hbm.at[pl.ds(base + (k - 2) * _CH, _CH)],
                                  osems[b]).wait()

        @plsc.parallel_loop(0, _CH, step=_L, unroll=_UNROLL)
        def gather_body(off, b=b):
            iv = idx_v[b, pl.ds(off, _L)]
            out_v[b, pl.ds(off, _L)] = plsc.load_gather(table_v, [iv])
        pltpu.async_copy(out_v.at[b], out_hbm.at[pl.ds(base + k * _CH, _CH)],
                         osems[b])

    pltpu.make_async_copy(out_v.at[(_NCH - 2) % 2],
                          out_hbm.at[pl.ds(base + (_NCH - 2) * _CH, _CH)],
                          osems[(_NCH - 2) % 2]).wait()
    pltpu.make_async_copy(out_v.at[(_NCH - 1) % 2],
                          out_hbm.at[pl.ds(base + (_NCH - 1) * _CH, _CH)],
                          osems[(_NCH - 1) % 2]).wait()


def kernel(table, idx):
    out = _sc_gather(table.reshape(-1), idx.reshape(-1))
    return out.reshape(idx.shape)


# speedup vs baseline: 353.7671x; 1.0016x over previous
import functools

import jax
import jax.numpy as jnp
from jax import lax
from jax.experimental import pallas as pl
from jax.experimental.pallas import tpu as pltpu
from jax.experimental.pallas import tpu_sc as plsc

_WIN = 32
_B = (_WIN * _WIN) ** 2
_TBL = (2 * _WIN - 1) ** 2
_NC, _NS, _L = 2, 16, 16
_NW = _NC * _NS
_BPW = _B // _NW
_CH = 8192
_NCH = _BPW // _CH
_UNROLL = 16


@functools.partial(
    pl.kernel,
    out_type=jax.ShapeDtypeStruct((_B,), jnp.float32),
    mesh=plsc.VectorSubcoreMesh(
        core_axis_name="c", subcore_axis_name="s",
        num_cores=_NC, num_subcores=_NS,
    ),
    compiler_params=pltpu.CompilerParams(needs_layout_passes=False),
    scratch_types=[
        pltpu.VMEM((_TBL,), jnp.float32),
        pltpu.VMEM((2, _CH), jnp.int32),
        pltpu.VMEM((2, _CH), jnp.float32),
        pltpu.SemaphoreType.DMA,
        pltpu.SemaphoreType.DMA,
        pltpu.SemaphoreType.DMA,
        pltpu.SemaphoreType.DMA,
    ],
)
def _sc_gather(table_hbm, idx_hbm, out_hbm, table_v, idx_v, out_v,
               isem0, isem1, osem0, osem1):
    isems = (isem0, isem1)
    osems = (osem0, osem1)
    wid = lax.axis_index("s") * _NC + lax.axis_index("c")
    base = wid * _BPW

    pltpu.async_copy(idx_hbm.at[pl.ds(base, _CH)], idx_v.at[0], isems[0])
    pltpu.sync_copy(table_hbm, table_v)

    for k in range(_NCH):
        b = k % 2
        if k + 1 < _NCH:
            pltpu.async_copy(idx_hbm.at[pl.ds(base + (k + 1) * _CH, _CH)],
                             idx_v.at[(k + 1) % 2], isems[(k + 1) % 2])
        pltpu.make_async_copy(idx_hbm.at[pl.ds(base + k * _CH, _CH)],
                              idx_v.at[b], isems[b]).wait()
        if k >= 2:
            pltpu.make_async_copy(out_v.at[b],
                                  out_hbm.at[pl.ds(base + (k - 2) * _CH, _CH)],
                                  osems[b]).wait()

        @plsc.parallel_loop(0, _CH, step=_L, unroll=_UNROLL)
        def gather_body(off, b=b):
            iv = idx_v[b, pl.ds(off, _L)]
            out_v[b, pl.ds(off, _L)] = plsc.load_gather(table_v, [iv])
        pltpu.async_copy(out_v.at[b], out_hbm.at[pl.ds(base + k * _CH, _CH)],
                         osems[b])

    pltpu.make_async_copy(out_v.at[(_NCH - 2) % 2],
                          out_hbm.at[pl.ds(base + (_NCH - 2) * _CH, _CH)],
                          osems[(_NCH - 2) % 2]).wait()
    pltpu.make_async_copy(out_v.at[(_NCH - 1) % 2],
                          out_hbm.at[pl.ds(base + (_NCH - 1) * _CH, _CH)],
                          osems[(_NCH - 1) % 2]).wait()


def kernel(table, idx):
    out = _sc_gather(table.reshape(-1), idx.reshape(-1))
    return out.reshape(idx.shape)


# speedup vs baseline: 429.4561x; 1.2140x over previous
import functools

import jax
import jax.numpy as jnp
from jax import lax
from jax.experimental import pallas as pl
from jax.experimental.pallas import tpu as pltpu
from jax.experimental.pallas import tpu_sc as plsc

_WIN = 32
_N = _WIN * _WIN
_TBL = (2 * _WIN - 1) ** 2
_NC, _NS, _L = 2, 16, 16
_NW = _NC * _NS
_RPW = _N // _NW
_CR = 8
_NCH = _RPW // _CR
_UNROLL = 8


@functools.partial(
    pl.kernel,
    out_type=jax.ShapeDtypeStruct((_N, _N), jnp.float32),
    mesh=plsc.VectorSubcoreMesh(
        core_axis_name="c", subcore_axis_name="s",
        num_cores=_NC, num_subcores=_NS,
    ),
    compiler_params=pltpu.CompilerParams(
        needs_layout_passes=False, use_tc_tiling_on_sc=True),
    scratch_types=[
        pltpu.VMEM((_TBL,), jnp.float32),
        pltpu.VMEM((2, _CR, _N), jnp.int32),
        pltpu.VMEM((2, _CR, _N), jnp.float32),
        pltpu.SemaphoreType.DMA,
        pltpu.SemaphoreType.DMA,
        pltpu.SemaphoreType.DMA,
        pltpu.SemaphoreType.DMA,
    ],
)
def _sc_gather(table_hbm, idx_hbm, out_hbm, table_v, idx_v, out_v,
               isem0, isem1, osem0, osem1):
    isems = (isem0, isem1)
    osems = (osem0, osem1)
    wid = lax.axis_index("s") * _NC + lax.axis_index("c")
    row0 = wid * _RPW

    pltpu.async_copy(idx_hbm.at[pl.ds(row0, _CR), :], idx_v.at[0], isems[0])
    pltpu.sync_copy(table_hbm, table_v)

    for k in range(_NCH):
        b = k % 2
        if k + 1 < _NCH:
            pltpu.async_copy(idx_hbm.at[pl.ds(row0 + (k + 1) * _CR, _CR), :],
                             idx_v.at[(k + 1) % 2], isems[(k + 1) % 2])
        pltpu.make_async_copy(idx_hbm.at[pl.ds(row0 + k * _CR, _CR), :],
                              idx_v.at[b], isems[b]).wait()
        if k >= 2:
            pltpu.make_async_copy(out_v.at[b],
                                  out_hbm.at[pl.ds(row0 + (k - 2) * _CR, _CR), :],
                                  osems[b]).wait()

        for r in range(_CR):
            @plsc.parallel_loop(0, _N, step=_L, unroll=_UNROLL)
            def gather_body(c, b=b, r=r):
                iv = idx_v[b, r, pl.ds(c, _L)]
                out_v[b, r, pl.ds(c, _L)] = plsc.load_gather(table_v, [iv])

        pltpu.async_copy(out_v.at[b],
                         out_hbm.at[pl.ds(row0 + k * _CR, _CR), :], osems[b])

    pltpu.make_async_copy(out_v.at[(_NCH - 2) % 2],
                          out_hbm.at[pl.ds(row0 + (_NCH - 2) * _CR, _CR), :],
                          osems[(_NCH - 2) % 2]).wait()
    pltpu.make_async_copy(out_v.at[(_NCH - 1) % 2],
                          out_hbm.at[pl.ds(row0 + (_NCH - 1) * _CR, _CR), :],
                          osems[(_NCH - 1) % 2]).wait()


def kernel(table, idx):
    return _sc_gather(table.reshape(-1), idx)
